# baseline (speedup 1.0000x reference)
import functools

import jax
import jax.numpy as jnp
from jax import lax
from jax.experimental import pallas as pl
from jax.experimental.pallas import tpu as pltpu
from jax.experimental.pallas import tpu_sc as plsc

B_, N_ = 4, 4096
IN_C, CGEO, N1, N2, KFP = 3, 256, 512, 128, 3
RADII1, NS1 = (0.1, 0.2, 0.4), (16, 32, 128)
RADII2, NS2 = (0.2, 0.4, 0.8), (32, 64, 128)
C1 = 64 + 128 + 128
C2 = 128 + 256 + 256

_NW = 32


def _sc_gather_call(table, idx, chunk, nchunks, qpw):
    D = table.shape[1]
    mesh = plsc.VectorSubcoreMesh(core_axis_name="c", subcore_axis_name="s")

    @functools.partial(
        pl.kernel, mesh=mesh,
        out_type=jax.ShapeDtypeStruct((idx.shape[0], D), jnp.float32),
        compiler_params=pltpu.CompilerParams(use_tc_tiling_on_sc=False),
        scratch_types=[
            pltpu.VMEM((chunk,), jnp.int32),
            pltpu.VMEM((chunk, D), jnp.float32),
            pltpu.SemaphoreType.DMA,
        ],
    )
    def k(table_hbm, idx_hbm, out_hbm, idx_v, rows_v, sem):
        wid = lax.axis_index("s") * 2 + lax.axis_index("c")
        base0 = wid * qpw

        def body(j, carry):
            base = base0 + j * chunk
            pltpu.sync_copy(idx_hbm.at[pl.ds(base, chunk)], idx_v)
            pltpu.async_copy(table_hbm.at[idx_v], rows_v, sem).wait()
            pltpu.sync_copy(rows_v, out_hbm.at[pl.ds(base, chunk)])
            return carry

        lax.fori_loop(0, nchunks, body, 0)

    return k(table, idx)


def _sc_gather(table, idx):
    Q = idx.shape[0]
    assert Q % _NW == 0
    qpw = Q // _NW
    chunk = 128
    while qpw % chunk:
        chunk //= 2
    return _sc_gather_call(table, idx, chunk, qpw // chunk, qpw)


def _pad16(a):
    d = a.shape[1]
    pad = (-d) % 16
    if pad:
        a = jnp.concatenate([a, jnp.zeros((a.shape[0], pad), a.dtype)], axis=1)
    return a


def _fps(pos_b, n_samples):
    dists = jnp.full((pos_b.shape[0],), jnp.inf, dtype=pos_b.dtype)
    idxs = jnp.zeros((n_samples,), dtype=jnp.int32)

    def body(i, carry):
        idxs, dists = carry
        d = jnp.sum((pos_b - pos_b[idxs[i - 1]]) ** 2, axis=1)
        dists = jnp.minimum(dists, d)
        return (idxs.at[i].set(jnp.argmax(dists).astype(jnp.int32)), dists)

    idxs, _ = jax.lax.fori_loop(1, n_samples, body, (idxs, dists))
    return idxs


def _gather(a, idx):
    return jax.vmap(lambda ab, ib: ab[ib])(a, idx)


def _apply_mlp_jax(layers, h, mask=None):
    red = tuple(range(h.ndim - 1))
    for lyr in layers:
        h = h @ lyr['W'].T + lyr['b']
        if mask is None:
            mean = h.mean(axis=red)
            var = ((h - mean) ** 2).mean(axis=red)
        else:
            m = mask[..., None].astype(h.dtype)
            cnt = jnp.maximum(mask.astype(h.dtype).sum(), 1.0)
            mean = (h * m).sum(axis=red) / cnt
            var = (((h - mean) ** 2) * m).sum(axis=red) / cnt
        h = (h - mean) / jnp.sqrt(var + 1e-5) * lyr['gamma'] + lyr['beta']
        h = jax.nn.relu(h)
    return h


def _msg_sa(x_flat, pos, pos_s, radii, nsamples, conv_params):
    B, N, _ = pos.shape
    M = pos_s.shape[1]
    C = x_flat.shape[1]
    d2 = jnp.sum((pos_s[:, :, None, :] - pos[:, None, :, :]) ** 2, axis=-1)
    pos_flat = pos.reshape(B * N, 3)
    pos_s_flat = pos_s.reshape(B * M, 3)
    x_self = x_flat[: B * M]
    rel_self = pos_flat[: B * M] - pos_s_flat
    msg_self = jnp.concatenate([x_self, rel_self], axis=1)[:, None, :]

    table = _pad16(jnp.concatenate([x_flat, pos_flat], axis=1))
    boff = (jnp.arange(B, dtype=jnp.int32) * N)[:, None, None]
    masks, nidxs = [], []
    for r, k in zip(radii, nsamples):
        neg, nidx = jax.lax.top_k(-d2, k)
        masks.append(((-neg) <= r * r).reshape(B * M, k))
        nidxs.append((nidx + boff).reshape(-1))
    rows = _sc_gather(table, jnp.concatenate(nidxs))
    splits = []
    o = 0
    for k in nsamples:
        splits.append(rows[o:o + B * M * k].reshape(B * M, k, table.shape[1]))
        o += B * M * k

    outs = []
    for r, k, layers, mask, rk in zip(radii, nsamples, conv_params, masks, splits):
        x_j = rk[:, :, :C]
        pos_j = rk[:, :, C:C + 3]
        rel = pos_j - pos_s_flat[:, None, :]
        msg = jnp.concatenate([x_j, rel], axis=2)
        msgs = jnp.concatenate([msg, msg_self], axis=1)
        mfull = jnp.concatenate([mask, jnp.ones((B * M, 1), bool)], axis=1)
        h = _apply_mlp_jax(layers, msgs, mfull)
        out = jnp.max(jnp.where(mfull[..., None], h, -jnp.inf), axis=1)
        outs.append(out)
    return jnp.concatenate(outs, axis=1)


def _knn_interp(x, pos_x, pos_y, k):
    B, nx, C = x.shape
    d2 = jnp.sum((pos_y[:, :, None, :] - pos_x[:, None, :, :]) ** 2, axis=-1)
    neg, idx = jax.lax.top_k(-d2, k)
    w = 1.0 / jnp.maximum(-neg, 1e-16)
    boff = (jnp.arange(B, dtype=jnp.int32) * nx)[:, None, None]
    flat = (idx + boff).reshape(-1)
    feats = _sc_gather(x.reshape(B * nx, C), flat).reshape(B, pos_y.shape[1], k, C)
    return (feats * w[..., None]).sum(axis=2) / w.sum(axis=2, keepdims=True)


def _mlp2_bn_kernel(x_ref, w1_ref, b1_ref, g1_ref, be1_ref, w2_ref, b2_ref,
                    g2_ref, be2_ref, out_ref):
    x = x_ref[...]
    h = jnp.dot(x, w1_ref[...].T, preferred_element_type=jnp.float32) + b1_ref[...]
    mean = jnp.mean(h, axis=0)
    var = jnp.mean((h - mean) ** 2, axis=0)
    h = (h - mean) * jax.lax.rsqrt(var + 1e-5) * g1_ref[...] + be1_ref[...]
    h = jnp.maximum(h, 0.0)
    h2 = jnp.dot(h, w2_ref[...].T, preferred_element_type=jnp.float32) + b2_ref[...]
    mean2 = jnp.mean(h2, axis=0)
    var2 = jnp.mean((h2 - mean2) ** 2, axis=0)
    h2 = (h2 - mean2) * jax.lax.rsqrt(var2 + 1e-5) * g2_ref[...] + be2_ref[...]
    out_ref[...] = jnp.maximum(h2, 0.0)


def _mlp2_bn(layers, x):
    l1, l2 = layers
    out_c = l2['W'].shape[0]
    return pl.pallas_call(
        _mlp2_bn_kernel,
        out_shape=jax.ShapeDtypeStruct((x.shape[0], out_c), jnp.float32),
    )(x, l1['W'], l1['b'], l1['gamma'], l1['beta'],
      l2['W'], l2['b'], l2['gamma'], l2['beta'])


def kernel(pts, params):
    B, N, _ = pts.shape
    pos = pts
    x0 = pts.reshape(B * N, 3)
    idx1 = jax.vmap(lambda p: _fps(p, N1))(pos)
    pos1 = _gather(pos, idx1)
    x1 = _msg_sa(x0, pos, pos1, RADII1, NS1, params['sa1'])
    idx2 = jax.vmap(lambda p: _fps(p, N2))(pos1)
    pos2 = _gather(pos1, idx2)
    x2 = _msg_sa(x1, pos1, pos2, RADII2, NS2, params['sa2'])
    g = _apply_mlp_jax(params['glob'], x2.reshape(B, N2, C2).max(axis=1))
    x1_up = _knn_interp(x2.reshape(B, N2, C2), pos2, pos1, KFP).reshape(B * N1, C2)
    x1_fp = _mlp2_bn(params['fp1'], jnp.concatenate([x1_up, x1], axis=1))
    x0_up = _knn_interp(x1_fp.reshape(B, N1, 256), pos1, pos, KFP).reshape(B * N, 256)
    F = _mlp2_bn(params['fp0'], jnp.concatenate([x0_up, x0], axis=1))
    return F.reshape(B, N, CGEO), g


# speedup vs baseline: 1.7458x; 1.7458x over previous
import functools

import jax
import jax.numpy as jnp
from jax import lax
from jax.experimental import pallas as pl
from jax.experimental.pallas import tpu as pltpu
from jax.experimental.pallas import tpu_sc as plsc

B_, N_ = 4, 4096
IN_C, CGEO, N1, N2, KFP = 3, 256, 512, 128, 3
RADII1, NS1 = (0.1, 0.2, 0.4), (16, 32, 128)
RADII2, NS2 = (0.2, 0.4, 0.8), (32, 64, 128)
C1 = 64 + 128 + 128
C2 = 128 + 256 + 256

_NW = 32


def _sc_gather_call(table, idx, chunk, nchunks, qpw):
    D = table.shape[1]
    mesh = plsc.VectorSubcoreMesh(core_axis_name="c", subcore_axis_name="s")

    @functools.partial(
        pl.kernel, mesh=mesh,
        out_type=jax.ShapeDtypeStruct((idx.shape[0], D), jnp.float32),
        compiler_params=pltpu.CompilerParams(use_tc_tiling_on_sc=False),
        scratch_types=[
            pltpu.VMEM((chunk,), jnp.int32),
            pltpu.VMEM((chunk, D), jnp.float32),
            pltpu.SemaphoreType.DMA,
        ],
    )
    def k(table_hbm, idx_hbm, out_hbm, idx_v, rows_v, sem):
        wid = lax.axis_index("s") * 2 + lax.axis_index("c")
        base0 = wid * qpw

        def body(j, carry):
            base = base0 + j * chunk
            pltpu.sync_copy(idx_hbm.at[pl.ds(base, chunk)], idx_v)
            pltpu.async_copy(table_hbm.at[idx_v], rows_v, sem).wait()
            pltpu.sync_copy(rows_v, out_hbm.at[pl.ds(base, chunk)])
            return carry

        lax.fori_loop(0, nchunks, body, 0)

    return k(table, idx)


def _sc_gather(table, idx):
    Q = idx.shape[0]
    assert Q % _NW == 0
    qpw = Q // _NW
    chunk = 128
    while qpw % chunk:
        chunk //= 2
    return _sc_gather_call(table, idx, chunk, qpw // chunk, qpw)


def _pad16(a):
    d = a.shape[1]
    pad = (-d) % 16
    if pad:
        a = jnp.concatenate([a, jnp.zeros((a.shape[0], pad), a.dtype)], axis=1)
    return a


def _fps_level(px, py, pz, n_samples):
    B, N = px.shape
    iota = lax.broadcasted_iota(jnp.int32, (B, N), 1)
    oiota = lax.broadcasted_iota(jnp.int32, (B, n_samples), 1)
    sx, sy, sz = px[:, 0:1], py[:, 0:1], pz[:, 0:1]
    idxb = jnp.zeros((B, n_samples), jnp.int32)
    ox = jnp.where(oiota == 0, sx, 0.0)
    oy = jnp.where(oiota == 0, sy, 0.0)
    oz = jnp.where(oiota == 0, sz, 0.0)
    dists = jnp.full((B, N), jnp.inf, jnp.float32)

    def body(i, c):
        dists, sx, sy, sz, idxb, ox, oy, oz = c
        d = (px - sx) ** 2 + (py - sy) ** 2 + (pz - sz) ** 2
        dists = jnp.minimum(dists, d)
        maxv = jnp.max(dists, axis=1, keepdims=True)
        nidx = jnp.min(jnp.where(dists == maxv, iota, N), axis=1, keepdims=True)
        sel = iota == nidx
        sx = jnp.sum(jnp.where(sel, px, 0.0), axis=1, keepdims=True)
        sy = jnp.sum(jnp.where(sel, py, 0.0), axis=1, keepdims=True)
        sz = jnp.sum(jnp.where(sel, pz, 0.0), axis=1, keepdims=True)
        rec = oiota == i
        idxb = jnp.where(rec, nidx, idxb)
        ox = jnp.where(rec, sx, ox)
        oy = jnp.where(rec, sy, oy)
        oz = jnp.where(rec, sz, oz)
        return (dists, sx, sy, sz, idxb, ox, oy, oz)

    c = (dists, sx, sy, sz, idxb, ox, oy, oz)
    c = lax.fori_loop(1, n_samples, body, c)
    _, _, _, _, idxb, ox, oy, oz = c
    return idxb, ox, oy, oz


def _fps_kernel_body(px_ref, py_ref, pz_ref, i1_ref, x1_ref, y1_ref, z1_ref,
                     i2_ref, x2_ref, y2_ref, z2_ref):
    idx1, ox, oy, oz = _fps_level(px_ref[...], py_ref[...], pz_ref[...], N1)
    i1_ref[...] = idx1
    x1_ref[...], y1_ref[...], z1_ref[...] = ox, oy, oz
    idx2, qx, qy, qz = _fps_level(ox, oy, oz, N2)
    i2_ref[...] = idx2
    x2_ref[...], y2_ref[...], z2_ref[...] = qx, qy, qz


def _fps_both(pos):
    B = pos.shape[0]
    px, py, pz = pos[:, :, 0], pos[:, :, 1], pos[:, :, 2]
    outs = pl.pallas_call(
        _fps_kernel_body,
        out_shape=[
            jax.ShapeDtypeStruct((B, N1), jnp.int32),
            jax.ShapeDtypeStruct((B, N1), jnp.float32),
            jax.ShapeDtypeStruct((B, N1), jnp.float32),
            jax.ShapeDtypeStruct((B, N1), jnp.float32),
            jax.ShapeDtypeStruct((B, N2), jnp.int32),
            jax.ShapeDtypeStruct((B, N2), jnp.float32),
            jax.ShapeDtypeStruct((B, N2), jnp.float32),
            jax.ShapeDtypeStruct((B, N2), jnp.float32),
        ],
    )(px, py, pz)
    idx1, x1, y1, z1, idx2, x2, y2, z2 = outs
    pos1 = jnp.stack([x1, y1, z1], axis=-1)
    pos2 = jnp.stack([x2, y2, z2], axis=-1)
    return idx1, pos1, idx2, pos2


def _gather(a, idx):
    return jax.vmap(lambda ab, ib: ab[ib])(a, idx)


def _apply_mlp_jax(layers, h, mask=None):
    red = tuple(range(h.ndim - 1))
    for lyr in layers:
        h = h @ lyr['W'].T + lyr['b']
        if mask is None:
            mean = h.mean(axis=red)
            var = ((h - mean) ** 2).mean(axis=red)
        else:
            m = mask[..., None].astype(h.dtype)
            cnt = jnp.maximum(mask.astype(h.dtype).sum(), 1.0)
            mean = (h * m).sum(axis=red) / cnt
            var = (((h - mean) ** 2) * m).sum(axis=red) / cnt
        h = (h - mean) / jnp.sqrt(var + 1e-5) * lyr['gamma'] + lyr['beta']
        h = jax.nn.relu(h)
    return h


def _msg_sa(x_flat, pos, pos_s, radii, nsamples, conv_params):
    B, N, _ = pos.shape
    M = pos_s.shape[1]
    C = x_flat.shape[1]
    d2 = jnp.sum((pos_s[:, :, None, :] - pos[:, None, :, :]) ** 2, axis=-1)
    pos_flat = pos.reshape(B * N, 3)
    pos_s_flat = pos_s.reshape(B * M, 3)
    x_self = x_flat[: B * M]
    rel_self = pos_flat[: B * M] - pos_s_flat
    msg_self = jnp.concatenate([x_self, rel_self], axis=1)[:, None, :]

    table = _pad16(jnp.concatenate([x_flat, pos_flat], axis=1))
    boff = (jnp.arange(B, dtype=jnp.int32) * N)[:, None, None]
    masks, nidxs = [], []
    for r, k in zip(radii, nsamples):
        neg, nidx = jax.lax.top_k(-d2, k)
        masks.append(((-neg) <= r * r).reshape(B * M, k))
        nidxs.append((nidx + boff).reshape(-1))
    rows = _sc_gather(table, jnp.concatenate(nidxs))
    splits = []
    o = 0
    for k in nsamples:
        splits.append(rows[o:o + B * M * k].reshape(B * M, k, table.shape[1]))
        o += B * M * k

    outs = []
    for r, k, layers, mask, rk in zip(radii, nsamples, conv_params, masks, splits):
        x_j = rk[:, :, :C]
        pos_j = rk[:, :, C:C + 3]
        rel = pos_j - pos_s_flat[:, None, :]
        msg = jnp.concatenate([x_j, rel], axis=2)
        msgs = jnp.concatenate([msg, msg_self], axis=1)
        mfull = jnp.concatenate([mask, jnp.ones((B * M, 1), bool)], axis=1)
        h = _apply_mlp_jax(layers, msgs, mfull)
        out = jnp.max(jnp.where(mfull[..., None], h, -jnp.inf), axis=1)
        outs.append(out)
    return jnp.concatenate(outs, axis=1)


def _knn_interp(x, pos_x, pos_y, k):
    B, nx, C = x.shape
    d2 = jnp.sum((pos_y[:, :, None, :] - pos_x[:, None, :, :]) ** 2, axis=-1)
    neg, idx = jax.lax.top_k(-d2, k)
    w = 1.0 / jnp.maximum(-neg, 1e-16)
    boff = (jnp.arange(B, dtype=jnp.int32) * nx)[:, None, None]
    flat = (idx + boff).reshape(-1)
    feats = _sc_gather(x.reshape(B * nx, C), flat).reshape(B, pos_y.shape[1], k, C)
    return (feats * w[..., None]).sum(axis=2) / w.sum(axis=2, keepdims=True)


def _mlp2_bn_kernel(x_ref, w1_ref, b1_ref, g1_ref, be1_ref, w2_ref, b2_ref,
                    g2_ref, be2_ref, out_ref):
    x = x_ref[...]
    h = jnp.dot(x, w1_ref[...].T, preferred_element_type=jnp.float32) + b1_ref[...]
    mean = jnp.mean(h, axis=0)
    var = jnp.mean((h - mean) ** 2, axis=0)
    h = (h - mean) * jax.lax.rsqrt(var + 1e-5) * g1_ref[...] + be1_ref[...]
    h = jnp.maximum(h, 0.0)
    h2 = jnp.dot(h, w2_ref[...].T, preferred_element_type=jnp.float32) + b2_ref[...]
    mean2 = jnp.mean(h2, axis=0)
    var2 = jnp.mean((h2 - mean2) ** 2, axis=0)
    h2 = (h2 - mean2) * jax.lax.rsqrt(var2 + 1e-5) * g2_ref[...] + be2_ref[...]
    out_ref[...] = jnp.maximum(h2, 0.0)


def _mlp2_bn(layers, x):
    l1, l2 = layers
    out_c = l2['W'].shape[0]
    return pl.pallas_call(
        _mlp2_bn_kernel,
        out_shape=jax.ShapeDtypeStruct((x.shape[0], out_c), jnp.float32),
    )(x, l1['W'], l1['b'], l1['gamma'], l1['beta'],
      l2['W'], l2['b'], l2['gamma'], l2['beta'])


def kernel(pts, params):
    B, N, _ = pts.shape
    pos = pts
    x0 = pts.reshape(B * N, 3)
    idx1, pos1, idx2, pos2 = _fps_both(pos)
    x1 = _msg_sa(x0, pos, pos1, RADII1, NS1, params['sa1'])
    x2 = _msg_sa(x1, pos1, pos2, RADII2, NS2, params['sa2'])
    g = _apply_mlp_jax(params['glob'], x2.reshape(B, N2, C2).max(axis=1))
    x1_up = _knn_interp(x2.reshape(B, N2, C2), pos2, pos1, KFP).reshape(B * N1, C2)
    x1_fp = _mlp2_bn(params['fp1'], jnp.concatenate([x1_up, x1], axis=1))
    x0_up = _knn_interp(x1_fp.reshape(B, N1, 256), pos1, pos, KFP).reshape(B * N, 256)
    F = _mlp2_bn(params['fp0'], jnp.concatenate([x0_up, x0], axis=1))
    return F.reshape(B, N, CGEO), g


# speedup vs baseline: 2.0043x; 1.1480x over previous
import functools

import jax
import jax.numpy as jnp
from jax import lax
from jax.experimental import pallas as pl
from jax.experimental.pallas import tpu as pltpu
from jax.experimental.pallas import tpu_sc as plsc

B_, N_ = 4, 4096
IN_C, CGEO, N1, N2, KFP = 3, 256, 512, 128, 3
RADII1, NS1 = (0.1, 0.2, 0.4), (16, 32, 128)
RADII2, NS2 = (0.2, 0.4, 0.8), (32, 64, 128)
C1 = 64 + 128 + 128
C2 = 128 + 256 + 256

_NW = 32


def _sc_gather_call(table, idx, chunk, nchunks, qpw):
    D = table.shape[1]
    mesh = plsc.VectorSubcoreMesh(core_axis_name="c", subcore_axis_name="s")

    @functools.partial(
        pl.kernel, mesh=mesh,
        out_type=jax.ShapeDtypeStruct((idx.shape[0], D), jnp.float32),
        compiler_params=pltpu.CompilerParams(use_tc_tiling_on_sc=False),
        scratch_types=[
            pltpu.VMEM((chunk,), jnp.int32),
            pltpu.VMEM((chunk, D), jnp.float32),
            pltpu.SemaphoreType.DMA,
        ],
    )
    def k(table_hbm, idx_hbm, out_hbm, idx_v, rows_v, sem):
        wid = lax.axis_index("s") * 2 + lax.axis_index("c")
        base0 = wid * qpw

        def body(j, carry):
            base = base0 + j * chunk
            pltpu.sync_copy(idx_hbm.at[pl.ds(base, chunk)], idx_v)
            pltpu.async_copy(table_hbm.at[idx_v], rows_v, sem).wait()
            pltpu.sync_copy(rows_v, out_hbm.at[pl.ds(base, chunk)])
            return carry

        lax.fori_loop(0, nchunks, body, 0)

    return k(table, idx)


def _sc_gather(table, idx):
    Q = idx.shape[0]
    assert Q % _NW == 0
    qpw = Q // _NW
    chunk = 128
    while qpw % chunk:
        chunk //= 2
    return _sc_gather_call(table, idx, chunk, qpw // chunk, qpw)


def _pad16(a):
    d = a.shape[1]
    pad = (-d) % 16
    if pad:
        a = jnp.concatenate([a, jnp.zeros((a.shape[0], pad), a.dtype)], axis=1)
    return a


def _fps_level(px, py, pz, n_samples):
    B, N = px.shape
    iota = lax.broadcasted_iota(jnp.int32, (B, N), 1)
    oiota = lax.broadcasted_iota(jnp.int32, (B, n_samples), 1)
    sx, sy, sz = px[:, 0:1], py[:, 0:1], pz[:, 0:1]
    idxb = jnp.zeros((B, n_samples), jnp.int32)
    ox = jnp.where(oiota == 0, sx, 0.0)
    oy = jnp.where(oiota == 0, sy, 0.0)
    oz = jnp.where(oiota == 0, sz, 0.0)
    dists = jnp.full((B, N), jnp.inf, jnp.float32)

    def body(i, c):
        dists, sx, sy, sz, idxb, ox, oy, oz = c
        d = (px - sx) ** 2 + (py - sy) ** 2 + (pz - sz) ** 2
        dists = jnp.minimum(dists, d)
        maxv = jnp.max(dists, axis=1, keepdims=True)
        nidx = jnp.min(jnp.where(dists == maxv, iota, N), axis=1, keepdims=True)
        sel = iota == nidx
        sx = jnp.sum(jnp.where(sel, px, 0.0), axis=1, keepdims=True)
        sy = jnp.sum(jnp.where(sel, py, 0.0), axis=1, keepdims=True)
        sz = jnp.sum(jnp.where(sel, pz, 0.0), axis=1, keepdims=True)
        rec = oiota == i
        idxb = jnp.where(rec, nidx, idxb)
        ox = jnp.where(rec, sx, ox)
        oy = jnp.where(rec, sy, oy)
        oz = jnp.where(rec, sz, oz)
        return (dists, sx, sy, sz, idxb, ox, oy, oz)

    c = (dists, sx, sy, sz, idxb, ox, oy, oz)
    c = lax.fori_loop(1, n_samples, body, c)
    _, _, _, _, idxb, ox, oy, oz = c
    return idxb, ox, oy, oz


def _fps_kernel_body(px_ref, py_ref, pz_ref, i1_ref, x1_ref, y1_ref, z1_ref,
                     i2_ref, x2_ref, y2_ref, z2_ref):
    idx1, ox, oy, oz = _fps_level(px_ref[...], py_ref[...], pz_ref[...], N1)
    i1_ref[...] = idx1
    x1_ref[...], y1_ref[...], z1_ref[...] = ox, oy, oz
    idx2, qx, qy, qz = _fps_level(ox, oy, oz, N2)
    i2_ref[...] = idx2
    x2_ref[...], y2_ref[...], z2_ref[...] = qx, qy, qz


def _fps_both(pos):
    B = pos.shape[0]
    px, py, pz = pos[:, :, 0], pos[:, :, 1], pos[:, :, 2]
    outs = pl.pallas_call(
        _fps_kernel_body,
        out_shape=[
            jax.ShapeDtypeStruct((B, N1), jnp.int32),
            jax.ShapeDtypeStruct((B, N1), jnp.float32),
            jax.ShapeDtypeStruct((B, N1), jnp.float32),
            jax.ShapeDtypeStruct((B, N1), jnp.float32),
            jax.ShapeDtypeStruct((B, N2), jnp.int32),
            jax.ShapeDtypeStruct((B, N2), jnp.float32),
            jax.ShapeDtypeStruct((B, N2), jnp.float32),
            jax.ShapeDtypeStruct((B, N2), jnp.float32),
        ],
    )(px, py, pz)
    idx1, x1, y1, z1, idx2, x2, y2, z2 = outs
    return (x1, y1, z1), (x2, y2, z2)


def _gather(a, idx):
    return jax.vmap(lambda ab, ib: ab[ib])(a, idx)


def _topk_body(k, n, sx_ref, sy_ref, sz_ref, px_ref, py_ref, pz_ref,
               oi_ref, ov_ref):
    rb = sx_ref.shape[2]
    sx = sx_ref[0, 0][:, None]
    sy = sy_ref[0, 0][:, None]
    sz = sz_ref[0, 0][:, None]
    px, py, pz = px_ref[0], py_ref[0], pz_ref[0]
    d2 = (sx - px) ** 2 + (sy - py) ** 2 + (sz - pz) ** 2
    iota = lax.broadcasted_iota(jnp.int32, (rb, n), 1)
    oio = lax.broadcasted_iota(jnp.int32, (rb, k), 1)

    def step(j, c):
        d2, vb, ib = c
        minv = jnp.min(d2, axis=1, keepdims=True)
        nidx = jnp.min(jnp.where(d2 == minv, iota, n), axis=1, keepdims=True)
        rec = oio == j
        vb = jnp.where(rec, minv, vb)
        ib = jnp.where(rec, nidx, ib)
        d2 = jnp.where(iota == nidx, jnp.inf, d2)
        return d2, vb, ib

    c = (d2, jnp.zeros((rb, k), jnp.float32), jnp.zeros((rb, k), jnp.int32))
    _, vb, ib = lax.fori_loop(0, k, step, c)
    oi_ref[0, 0] = ib
    ov_ref[0, 0] = vb


def _topk_ext(s_planes, p_planes, k, rb):
    sx, sy, sz = s_planes
    px, py, pz = p_planes
    B, M = sx.shape
    N = px.shape[1]
    G = M // rb
    sx, sy, sz = (a.reshape(B * G, 1, rb) for a in (sx, sy, sz))
    px, py, pz = (a.reshape(B, 1, N) for a in (px, py, pz))
    sspec = pl.BlockSpec((1, 1, rb), lambda i: (i, 0, 0))
    pspec = pl.BlockSpec((1, 1, N), lambda i: (i // G, 0, 0))
    ospec = pl.BlockSpec((1, 1, rb, k), lambda i: (i, 0, 0, 0))
    oi, ov = pl.pallas_call(
        functools.partial(_topk_body, k, N),
        grid=(B * G,),
        in_specs=[sspec, sspec, sspec, pspec, pspec, pspec],
        out_specs=[ospec, ospec],
        out_shape=[
            jax.ShapeDtypeStruct((B * G, 1, rb, k), jnp.int32),
            jax.ShapeDtypeStruct((B * G, 1, rb, k), jnp.float32),
        ],
    )(sx, sy, sz, px, py, pz)
    return oi.reshape(B, M, k), ov.reshape(B, M, k)


def _apply_mlp_jax(layers, h, mask=None):
    red = tuple(range(h.ndim - 1))
    for lyr in layers:
        h = h @ lyr['W'].T + lyr['b']
        if mask is None:
            mean = h.mean(axis=red)
            var = ((h - mean) ** 2).mean(axis=red)
        else:
            m = mask[..., None].astype(h.dtype)
            cnt = jnp.maximum(mask.astype(h.dtype).sum(), 1.0)
            mean = (h * m).sum(axis=red) / cnt
            var = (((h - mean) ** 2) * m).sum(axis=red) / cnt
        h = (h - mean) / jnp.sqrt(var + 1e-5) * lyr['gamma'] + lyr['beta']
        h = jax.nn.relu(h)
    return h


def _msg_sa(x_flat, pos, pos_s, radii, nsamples, conv_params, tidx, tval):
    B, N, _ = pos.shape
    M = pos_s.shape[1]
    C = x_flat.shape[1]
    pos_flat = pos.reshape(B * N, 3)
    pos_s_flat = pos_s.reshape(B * M, 3)
    x_self = x_flat[: B * M]
    rel_self = pos_flat[: B * M] - pos_s_flat
    msg_self = jnp.concatenate([x_self, rel_self], axis=1)[:, None, :]

    table = _pad16(jnp.concatenate([x_flat, pos_flat], axis=1))
    boff = (jnp.arange(B, dtype=jnp.int32) * N)[:, None, None]
    masks, nidxs = [], []
    for r, k in zip(radii, nsamples):
        masks.append((tval[:, :, :k] <= r * r).reshape(B * M, k))
        nidxs.append((tidx[:, :, :k] + boff).reshape(-1))
    rows = _sc_gather(table, jnp.concatenate(nidxs))
    splits = []
    o = 0
    for k in nsamples:
        splits.append(rows[o:o + B * M * k].reshape(B * M, k, table.shape[1]))
        o += B * M * k

    outs = []
    for r, k, layers, mask, rk in zip(radii, nsamples, conv_params, masks, splits):
        x_j = rk[:, :, :C]
        pos_j = rk[:, :, C:C + 3]
        rel = pos_j - pos_s_flat[:, None, :]
        msg = jnp.concatenate([x_j, rel], axis=2)
        msgs = jnp.concatenate([msg, msg_self], axis=1)
        mfull = jnp.concatenate([mask, jnp.ones((B * M, 1), bool)], axis=1)
        h = _apply_mlp_jax(layers, msgs, mfull)
        out = jnp.max(jnp.where(mfull[..., None], h, -jnp.inf), axis=1)
        outs.append(out)
    return jnp.concatenate(outs, axis=1)


def _knn_interp(x, idx, d2v):
    B, nx, C = x.shape
    k = idx.shape[2]
    w = 1.0 / jnp.maximum(d2v, 1e-16)
    boff = (jnp.arange(B, dtype=jnp.int32) * nx)[:, None, None]
    flat = (idx + boff).reshape(-1)
    feats = _sc_gather(x.reshape(B * nx, C), flat).reshape(B, idx.shape[1], k, C)
    return (feats * w[..., None]).sum(axis=2) / w.sum(axis=2, keepdims=True)


def _mlp2_bn_kernel(x_ref, w1_ref, b1_ref, g1_ref, be1_ref, w2_ref, b2_ref,
                    g2_ref, be2_ref, out_ref):
    x = x_ref[...]
    h = jnp.dot(x, w1_ref[...].T, preferred_element_type=jnp.float32) + b1_ref[...]
    mean = jnp.mean(h, axis=0)
    var = jnp.mean((h - mean) ** 2, axis=0)
    h = (h - mean) * jax.lax.rsqrt(var + 1e-5) * g1_ref[...] + be1_ref[...]
    h = jnp.maximum(h, 0.0)
    h2 = jnp.dot(h, w2_ref[...].T, preferred_element_type=jnp.float32) + b2_ref[...]
    mean2 = jnp.mean(h2, axis=0)
    var2 = jnp.mean((h2 - mean2) ** 2, axis=0)
    h2 = (h2 - mean2) * jax.lax.rsqrt(var2 + 1e-5) * g2_ref[...] + be2_ref[...]
    out_ref[...] = jnp.maximum(h2, 0.0)


def _mlp2_bn(layers, x):
    l1, l2 = layers
    out_c = l2['W'].shape[0]
    return pl.pallas_call(
        _mlp2_bn_kernel,
        out_shape=jax.ShapeDtypeStruct((x.shape[0], out_c), jnp.float32),
    )(x, l1['W'], l1['b'], l1['gamma'], l1['beta'],
      l2['W'], l2['b'], l2['gamma'], l2['beta'])


def kernel(pts, params):
    B, N, _ = pts.shape
    pos = pts
    x0 = pts.reshape(B * N, 3)
    p0 = (pos[:, :, 0], pos[:, :, 1], pos[:, :, 2])
    p1, p2 = _fps_both(pos)
    pos1 = jnp.stack(p1, axis=-1)
    pos2 = jnp.stack(p2, axis=-1)
    ti1, tv1 = _topk_ext(p1, p0, 128, 128)
    x1 = _msg_sa(x0, pos, pos1, RADII1, NS1, params['sa1'], ti1, tv1)
    ti2, tv2 = _topk_ext(p2, p1, 128, 128)
    x2 = _msg_sa(x1, pos1, pos2, RADII2, NS2, params['sa2'], ti2, tv2)
    g = _apply_mlp_jax(params['glob'], x2.reshape(B, N2, C2).max(axis=1))
    ki1, kv1 = _topk_ext(p1, p2, KFP, 128)
    x1_up = _knn_interp(x2.reshape(B, N2, C2), ki1, kv1).reshape(B * N1, C2)
    x1_fp = _mlp2_bn(params['fp1'], jnp.concatenate([x1_up, x1], axis=1))
    ki0, kv0 = _topk_ext(p0, p1, KFP, 512)
    x0_up = _knn_interp(x1_fp.reshape(B, N1, 256), ki0, kv0).reshape(B * N, 256)
    F = _mlp2_bn(params['fp0'], jnp.concatenate([x0_up, x0], axis=1))
    return F.reshape(B, N, CGEO), g


# speedup vs baseline: 2.0169x; 1.0063x over previous
import functools

import jax
import jax.numpy as jnp
from jax import lax
from jax.experimental import pallas as pl
from jax.experimental.pallas import tpu as pltpu
from jax.experimental.pallas import tpu_sc as plsc

B_, N_ = 4, 4096
IN_C, CGEO, N1, N2, KFP = 3, 256, 512, 128, 3
RADII1, NS1 = (0.1, 0.2, 0.4), (16, 32, 128)
RADII2, NS2 = (0.2, 0.4, 0.8), (32, 64, 128)
C1 = 64 + 128 + 128
C2 = 128 + 256 + 256

_NW = 32


def _sc_gather_call(table, idx, chunk, nchunks, qpw):
    D = table.shape[1]
    npairs = nchunks // 2
    tail = nchunks % 2
    mesh = plsc.VectorSubcoreMesh(core_axis_name="c", subcore_axis_name="s")

    @functools.partial(
        pl.kernel, mesh=mesh,
        out_type=jax.ShapeDtypeStruct((idx.shape[0], D), jnp.float32),
        compiler_params=pltpu.CompilerParams(use_tc_tiling_on_sc=False),
        scratch_types=[
            pltpu.VMEM((chunk,), jnp.int32),
            pltpu.VMEM((chunk,), jnp.int32),
            pltpu.VMEM((chunk, D), jnp.float32),
            pltpu.VMEM((chunk, D), jnp.float32),
            pltpu.SemaphoreType.DMA,
            pltpu.SemaphoreType.DMA,
            pltpu.SemaphoreType.DMA,
            pltpu.SemaphoreType.DMA,
        ],
    )
    def k(table_hbm, idx_hbm, out_hbm, idx0, idx1, rows0, rows1,
          sg0, sg1, sw0, sw1):
        wid = lax.axis_index("s") * 2 + lax.axis_index("c")
        base0 = wid * qpw
        idx_v = (idx0, idx1)
        rows_v = (rows0, rows1)
        sg = (sg0, sg1)
        sw = (sw0, sw1)

        def pair(jj, carry):
            handles = []
            for p in (0, 1):
                base = base0 + (jj * 2 + p) * chunk

                @pl.when(jj > 0)
                def _wait_wb():
                    pltpu.make_async_copy(
                        rows_v[p], out_hbm.at[pl.ds(base0, chunk)], sw[p]
                    ).wait()

                pltpu.sync_copy(idx_hbm.at[pl.ds(base, chunk)], idx_v[p])
                handles.append(
                    pltpu.async_copy(table_hbm.at[idx_v[p]], rows_v[p], sg[p]))
            for p in (0, 1):
                base = base0 + (jj * 2 + p) * chunk
                handles[p].wait()
                pltpu.async_copy(rows_v[p], out_hbm.at[pl.ds(base, chunk)],
                                 sw[p])
            return carry

        if npairs:
            lax.fori_loop(0, npairs, pair, 0)
            for p in (0, 1):
                pltpu.make_async_copy(
                    rows_v[p], out_hbm.at[pl.ds(base0, chunk)], sw[p]
                ).wait()
        if tail:
            base = base0 + (nchunks - 1) * chunk
            pltpu.sync_copy(idx_hbm.at[pl.ds(base, chunk)], idx0)
            pltpu.async_copy(table_hbm.at[idx0], rows0, sg0).wait()
            pltpu.sync_copy(rows0, out_hbm.at[pl.ds(base, chunk)])

    return k(table, idx)


def _sc_gather(table, idx):
    Q = idx.shape[0]
    assert Q % _NW == 0
    qpw = Q // _NW
    chunk = 128
    while chunk * table.shape[1] * 4 > 200_000:
        chunk //= 2
    while qpw % chunk:
        chunk //= 2
    return _sc_gather_call(table, idx, chunk, qpw // chunk, qpw)


def _pad16(a):
    d = a.shape[1]
    pad = (-d) % 16
    if pad:
        a = jnp.concatenate([a, jnp.zeros((a.shape[0], pad), a.dtype)], axis=1)
    return a


def _fps_level(px, py, pz, n_samples):
    B, N = px.shape
    iota = lax.broadcasted_iota(jnp.int32, (B, N), 1)
    oiota = lax.broadcasted_iota(jnp.int32, (B, n_samples), 1)
    sx, sy, sz = px[:, 0:1], py[:, 0:1], pz[:, 0:1]
    idxb = jnp.zeros((B, n_samples), jnp.int32)
    ox = jnp.where(oiota == 0, sx, 0.0)
    oy = jnp.where(oiota == 0, sy, 0.0)
    oz = jnp.where(oiota == 0, sz, 0.0)
    dists = jnp.full((B, N), jnp.inf, jnp.float32)

    def body(i, c):
        dists, sx, sy, sz, idxb, ox, oy, oz = c
        d = (px - sx) ** 2 + (py - sy) ** 2 + (pz - sz) ** 2
        dists = jnp.minimum(dists, d)
        maxv = jnp.max(dists, axis=1, keepdims=True)
        nidx = jnp.min(jnp.where(dists == maxv, iota, N), axis=1, keepdims=True)
        sel = iota == nidx
        sx = jnp.sum(jnp.where(sel, px, 0.0), axis=1, keepdims=True)
        sy = jnp.sum(jnp.where(sel, py, 0.0), axis=1, keepdims=True)
        sz = jnp.sum(jnp.where(sel, pz, 0.0), axis=1, keepdims=True)
        rec = oiota == i
        idxb = jnp.where(rec, nidx, idxb)
        ox = jnp.where(rec, sx, ox)
        oy = jnp.where(rec, sy, oy)
        oz = jnp.where(rec, sz, oz)
        return (dists, sx, sy, sz, idxb, ox, oy, oz)

    c = (dists, sx, sy, sz, idxb, ox, oy, oz)
    c = lax.fori_loop(1, n_samples, body, c)
    _, _, _, _, idxb, ox, oy, oz = c
    return idxb, ox, oy, oz


def _fps_kernel_body(px_ref, py_ref, pz_ref, i1_ref, x1_ref, y1_ref, z1_ref,
                     i2_ref, x2_ref, y2_ref, z2_ref):
    idx1, ox, oy, oz = _fps_level(px_ref[...], py_ref[...], pz_ref[...], N1)
    i1_ref[...] = idx1
    x1_ref[...], y1_ref[...], z1_ref[...] = ox, oy, oz
    idx2, qx, qy, qz = _fps_level(ox, oy, oz, N2)
    i2_ref[...] = idx2
    x2_ref[...], y2_ref[...], z2_ref[...] = qx, qy, qz


def _fps_both(pos):
    B = pos.shape[0]
    px, py, pz = pos[:, :, 0], pos[:, :, 1], pos[:, :, 2]
    outs = pl.pallas_call(
        _fps_kernel_body,
        out_shape=[
            jax.ShapeDtypeStruct((B, N1), jnp.int32),
            jax.ShapeDtypeStruct((B, N1), jnp.float32),
            jax.ShapeDtypeStruct((B, N1), jnp.float32),
            jax.ShapeDtypeStruct((B, N1), jnp.float32),
            jax.ShapeDtypeStruct((B, N2), jnp.int32),
            jax.ShapeDtypeStruct((B, N2), jnp.float32),
            jax.ShapeDtypeStruct((B, N2), jnp.float32),
            jax.ShapeDtypeStruct((B, N2), jnp.float32),
        ],
    )(px, py, pz)
    idx1, x1, y1, z1, idx2, x2, y2, z2 = outs
    return (x1, y1, z1), (x2, y2, z2)


def _gather(a, idx):
    return jax.vmap(lambda ab, ib: ab[ib])(a, idx)


def _topk_body(k, n, sx_ref, sy_ref, sz_ref, px_ref, py_ref, pz_ref,
               oi_ref, ov_ref):
    rb = sx_ref.shape[2]
    sx = sx_ref[0, 0][:, None]
    sy = sy_ref[0, 0][:, None]
    sz = sz_ref[0, 0][:, None]
    px, py, pz = px_ref[0], py_ref[0], pz_ref[0]
    d2 = (sx - px) ** 2 + (sy - py) ** 2 + (sz - pz) ** 2
    iota = lax.broadcasted_iota(jnp.int32, (rb, n), 1)
    oio = lax.broadcasted_iota(jnp.int32, (rb, k), 1)

    def step(j, c):
        d2, vb, ib = c
        minv = jnp.min(d2, axis=1, keepdims=True)
        nidx = jnp.min(jnp.where(d2 == minv, iota, n), axis=1, keepdims=True)
        rec = oio == j
        vb = jnp.where(rec, minv, vb)
        ib = jnp.where(rec, nidx, ib)
        d2 = jnp.where(iota == nidx, jnp.inf, d2)
        return d2, vb, ib

    c = (d2, jnp.zeros((rb, k), jnp.float32), jnp.zeros((rb, k), jnp.int32))
    _, vb, ib = lax.fori_loop(0, k, step, c)
    oi_ref[0, 0] = ib
    ov_ref[0, 0] = vb


def _topk_ext(s_planes, p_planes, k, rb):
    sx, sy, sz = s_planes
    px, py, pz = p_planes
    B, M = sx.shape
    N = px.shape[1]
    G = M // rb
    sx, sy, sz = (a.reshape(B * G, 1, rb) for a in (sx, sy, sz))
    px, py, pz = (a.reshape(B, 1, N) for a in (px, py, pz))
    sspec = pl.BlockSpec((1, 1, rb), lambda i: (i, 0, 0))
    pspec = pl.BlockSpec((1, 1, N), lambda i: (i // G, 0, 0))
    ospec = pl.BlockSpec((1, 1, rb, k), lambda i: (i, 0, 0, 0))
    oi, ov = pl.pallas_call(
        functools.partial(_topk_body, k, N),
        grid=(B * G,),
        in_specs=[sspec, sspec, sspec, pspec, pspec, pspec],
        out_specs=[ospec, ospec],
        out_shape=[
            jax.ShapeDtypeStruct((B * G, 1, rb, k), jnp.int32),
            jax.ShapeDtypeStruct((B * G, 1, rb, k), jnp.float32),
        ],
    )(sx, sy, sz, px, py, pz)
    return oi.reshape(B, M, k), ov.reshape(B, M, k)


def _apply_mlp_jax(layers, h, mask=None):
    red = tuple(range(h.ndim - 1))
    for lyr in layers:
        h = h @ lyr['W'].T + lyr['b']
        if mask is None:
            mean = h.mean(axis=red)
            var = ((h - mean) ** 2).mean(axis=red)
        else:
            m = mask[..., None].astype(h.dtype)
            cnt = jnp.maximum(mask.astype(h.dtype).sum(), 1.0)
            mean = (h * m).sum(axis=red) / cnt
            var = (((h - mean) ** 2) * m).sum(axis=red) / cnt
        h = (h - mean) / jnp.sqrt(var + 1e-5) * lyr['gamma'] + lyr['beta']
        h = jax.nn.relu(h)
    return h


def _msg_sa(x_flat, pos, pos_s, radii, nsamples, conv_params, tidx, tval):
    B, N, _ = pos.shape
    M = pos_s.shape[1]
    C = x_flat.shape[1]
    pos_flat = pos.reshape(B * N, 3)
    pos_s_flat = pos_s.reshape(B * M, 3)
    x_self = x_flat[: B * M]
    rel_self = pos_flat[: B * M] - pos_s_flat
    msg_self = jnp.concatenate([x_self, rel_self], axis=1)[:, None, :]

    table = _pad16(jnp.concatenate([x_flat, pos_flat], axis=1))
    boff = (jnp.arange(B, dtype=jnp.int32) * N)[:, None, None]
    masks, nidxs = [], []
    for r, k in zip(radii, nsamples):
        masks.append((tval[:, :, :k] <= r * r).reshape(B * M, k))
        nidxs.append((tidx[:, :, :k] + boff).reshape(-1))
    rows = _sc_gather(table, jnp.concatenate(nidxs))
    splits = []
    o = 0
    for k in nsamples:
        splits.append(rows[o:o + B * M * k].reshape(B * M, k, table.shape[1]))
        o += B * M * k

    outs = []
    for r, k, layers, mask, rk in zip(radii, nsamples, conv_params, masks, splits):
        x_j = rk[:, :, :C]
        pos_j = rk[:, :, C:C + 3]
        rel = pos_j - pos_s_flat[:, None, :]
        msg = jnp.concatenate([x_j, rel], axis=2)
        msgs = jnp.concatenate([msg, msg_self], axis=1)
        mfull = jnp.concatenate([mask, jnp.ones((B * M, 1), bool)], axis=1)
        h = _apply_mlp_jax(layers, msgs, mfull)
        out = jnp.max(jnp.where(mfull[..., None], h, -jnp.inf), axis=1)
        outs.append(out)
    return jnp.concatenate(outs, axis=1)


def _knn_interp(x, idx, d2v):
    B, nx, C = x.shape
    k = idx.shape[2]
    w = 1.0 / jnp.maximum(d2v, 1e-16)
    boff = (jnp.arange(B, dtype=jnp.int32) * nx)[:, None, None]
    flat = (idx + boff).reshape(-1)
    feats = _sc_gather(x.reshape(B * nx, C), flat).reshape(B, idx.shape[1], k, C)
    return (feats * w[..., None]).sum(axis=2) / w.sum(axis=2, keepdims=True)


def _mlp2_bn_kernel(x_ref, w1_ref, b1_ref, g1_ref, be1_ref, w2_ref, b2_ref,
                    g2_ref, be2_ref, out_ref):
    x = x_ref[...]
    h = jnp.dot(x, w1_ref[...].T, preferred_element_type=jnp.float32) + b1_ref[...]
    mean = jnp.mean(h, axis=0)
    var = jnp.mean((h - mean) ** 2, axis=0)
    h = (h - mean) * jax.lax.rsqrt(var + 1e-5) * g1_ref[...] + be1_ref[...]
    h = jnp.maximum(h, 0.0)
    h2 = jnp.dot(h, w2_ref[...].T, preferred_element_type=jnp.float32) + b2_ref[...]
    mean2 = jnp.mean(h2, axis=0)
    var2 = jnp.mean((h2 - mean2) ** 2, axis=0)
    h2 = (h2 - mean2) * jax.lax.rsqrt(var2 + 1e-5) * g2_ref[...] + be2_ref[...]
    out_ref[...] = jnp.maximum(h2, 0.0)


def _mlp2_bn(layers, x):
    l1, l2 = layers
    out_c = l2['W'].shape[0]
    return pl.pallas_call(
        _mlp2_bn_kernel,
        out_shape=jax.ShapeDtypeStruct((x.shape[0], out_c), jnp.float32),
    )(x, l1['W'], l1['b'], l1['gamma'], l1['beta'],
      l2['W'], l2['b'], l2['gamma'], l2['beta'])


def kernel(pts, params):
    B, N, _ = pts.shape
    pos = pts
    x0 = pts.reshape(B * N, 3)
    p0 = (pos[:, :, 0], pos[:, :, 1], pos[:, :, 2])
    p1, p2 = _fps_both(pos)
    pos1 = jnp.stack(p1, axis=-1)
    pos2 = jnp.stack(p2, axis=-1)
    ti1, tv1 = _topk_ext(p1, p0, 128, 128)
    x1 = _msg_sa(x0, pos, pos1, RADII1, NS1, params['sa1'], ti1, tv1)
    ti2, tv2 = _topk_ext(p2, p1, 128, 128)
    x2 = _msg_sa(x1, pos1, pos2, RADII2, NS2, params['sa2'], ti2, tv2)
    g = _apply_mlp_jax(params['glob'], x2.reshape(B, N2, C2).max(axis=1))
    ki1, kv1 = _topk_ext(p1, p2, KFP, 128)
    x1_up = _knn_interp(x2.reshape(B, N2, C2), ki1, kv1).reshape(B * N1, C2)
    x1_fp = _mlp2_bn(params['fp1'], jnp.concatenate([x1_up, x1], axis=1))
    ki0, kv0 = _topk_ext(p0, p1, KFP, 512)
    x0_up = _knn_interp(x1_fp.reshape(B, N1, 256), ki0, kv0).reshape(B * N, 256)
    F = _mlp2_bn(params['fp0'], jnp.concatenate([x0_up, x0], axis=1))
    return F.reshape(B, N, CGEO), g
